# Initial kernel scaffold; baseline (speedup 1.0000x reference)
import jax
import jax.numpy as jnp
from jax.experimental import pallas as pl


def kernel(x, edge_index, W1, att_src1, att_dst1, b1, W2, att_src2, att_dst2, b2):
    raise NotImplementedError("write your pallas kernel here")


# speedup vs baseline: 2.7502x; 2.7502x over previous
import jax
import jax.numpy as jnp
from jax.experimental import pallas as pl

_N = 10000
_E = 160000
_IN = 256
_HID = 256
_HEADS = 4
_OUT = 128


def _proj1_body(x_ref, w_ref, as_ref, ad_ref, h_ref, asrc_ref, adst_ref):
    h = jnp.dot(x_ref[...], w_ref[...], preferred_element_type=jnp.float32)
    h_ref[...] = h
    asrc_ref[...] = jnp.dot(h, as_ref[...], preferred_element_type=jnp.float32)
    adst_ref[...] = jnp.dot(h, ad_ref[...], preferred_element_type=jnp.float32)


def _proj1(x, w_t, a_src_mat, a_dst_mat, block_n):
    n, k = x.shape
    f = w_t.shape[1]
    hh = a_src_mat.shape[1]
    return pl.pallas_call(
        _proj1_body,
        grid=(n // block_n,),
        in_specs=[
            pl.BlockSpec((block_n, k), lambda i: (i, 0)),
            pl.BlockSpec((k, f), lambda i: (0, 0)),
            pl.BlockSpec((f, hh), lambda i: (0, 0)),
            pl.BlockSpec((f, hh), lambda i: (0, 0)),
        ],
        out_specs=[
            pl.BlockSpec((block_n, f), lambda i: (i, 0)),
            pl.BlockSpec((block_n, hh), lambda i: (i, 0)),
            pl.BlockSpec((block_n, hh), lambda i: (i, 0)),
        ],
        out_shape=[
            jax.ShapeDtypeStruct((n, f), jnp.float32),
            jax.ShapeDtypeStruct((n, hh), jnp.float32),
            jax.ShapeDtypeStruct((n, hh), jnp.float32),
        ],
    )(x, w_t, a_src_mat, a_dst_mat)


def _proj2_body(y_ref, b1_ref, w_ref, as_ref, ad_ref, g_ref, asrc_ref, adst_ref):
    y0 = y_ref[...] + b1_ref[...]
    y = jnp.where(y0 > 0.0, y0, jnp.exp(jnp.minimum(y0, 0.0)) - 1.0)
    g = jnp.dot(y, w_ref[...], preferred_element_type=jnp.float32)
    g_ref[...] = g
    asrc_ref[...] = jnp.dot(g, as_ref[...], preferred_element_type=jnp.float32)
    adst_ref[...] = jnp.dot(g, ad_ref[...], preferred_element_type=jnp.float32)


def _proj2(y, b1, w_t, a_src_mat, a_dst_mat, block_n):
    n, k = y.shape
    f = w_t.shape[1]
    hh = a_src_mat.shape[1]
    return pl.pallas_call(
        _proj2_body,
        grid=(n // block_n,),
        in_specs=[
            pl.BlockSpec((block_n, k), lambda i: (i, 0)),
            pl.BlockSpec((1, k), lambda i: (0, 0)),
            pl.BlockSpec((k, f), lambda i: (0, 0)),
            pl.BlockSpec((f, hh), lambda i: (0, 0)),
            pl.BlockSpec((f, hh), lambda i: (0, 0)),
        ],
        out_specs=[
            pl.BlockSpec((block_n, f), lambda i: (i, 0)),
            pl.BlockSpec((block_n, hh), lambda i: (i, 0)),
            pl.BlockSpec((block_n, hh), lambda i: (i, 0)),
        ],
        out_shape=[
            jax.ShapeDtypeStruct((n, f), jnp.float32),
            jax.ShapeDtypeStruct((n, hh), jnp.float32),
            jax.ShapeDtypeStruct((n, hh), jnp.float32),
        ],
    )(y, b1, w_t, a_src_mat, a_dst_mat)


def _logits_body(es_ref, ed_ref, o_ref):
    s = es_ref[...] + ed_ref[...]
    o_ref[...] = jnp.where(s >= 0.0, s, 0.2 * s)


def _edge_logits(e_src, e_dst, block_e):
    e, hh = e_src.shape
    return pl.pallas_call(
        _logits_body,
        grid=(e // block_e,),
        in_specs=[
            pl.BlockSpec((block_e, hh), lambda i: (i, 0)),
            pl.BlockSpec((block_e, hh), lambda i: (i, 0)),
        ],
        out_specs=pl.BlockSpec((block_e, hh), lambda i: (i, 0)),
        out_shape=jax.ShapeDtypeStruct((e, hh), jnp.float32),
    )(e_src, e_dst)


def _exp_body(e_ref, m_ref, o_ref):
    o_ref[...] = jnp.exp(e_ref[...] - m_ref[...])


def _edge_exp(e, m_g, block_e):
    ee, hh = e.shape
    return pl.pallas_call(
        _exp_body,
        grid=(ee // block_e,),
        in_specs=[
            pl.BlockSpec((block_e, hh), lambda i: (i, 0)),
            pl.BlockSpec((block_e, hh), lambda i: (i, 0)),
        ],
        out_specs=pl.BlockSpec((block_e, hh), lambda i: (i, 0)),
        out_shape=jax.ShapeDtypeStruct((ee, hh), jnp.float32),
    )(e, m_g)


def _msg_body(h_ref, ex_ref, den_ref, o_ref):
    alpha = ex_ref[...] / (den_ref[...] + 1e-16)
    h = h_ref[...]
    b, f = h.shape
    hh = alpha.shape[1]
    c = f // hh
    parts = [jnp.broadcast_to(alpha[:, j:j + 1], (b, c)) for j in range(hh)]
    o_ref[...] = h * jnp.concatenate(parts, axis=1)


def _edge_msg(h_src, ex, den_g, block_e):
    ee, f = h_src.shape
    hh = ex.shape[1]
    return pl.pallas_call(
        _msg_body,
        grid=(ee // block_e,),
        in_specs=[
            pl.BlockSpec((block_e, f), lambda i: (i, 0)),
            pl.BlockSpec((block_e, hh), lambda i: (i, 0)),
            pl.BlockSpec((block_e, hh), lambda i: (i, 0)),
        ],
        out_specs=pl.BlockSpec((block_e, f), lambda i: (i, 0)),
        out_shape=jax.ShapeDtypeStruct((ee, f), jnp.float32),
    )(h_src, ex, den_g)


def _lsm_body(z_ref, b_ref, o_ref):
    z = z_ref[...] + b_ref[...]
    m = jnp.max(z, axis=1, keepdims=True)
    ez = jnp.exp(z - m)
    s = jnp.sum(ez, axis=1, keepdims=True)
    o_ref[...] = z - m - jnp.log(s)


def _log_softmax(z, b, block_n):
    n, f = z.shape
    return pl.pallas_call(
        _lsm_body,
        grid=(n // block_n,),
        in_specs=[
            pl.BlockSpec((block_n, f), lambda i: (i, 0)),
            pl.BlockSpec((1, f), lambda i: (0, 0)),
        ],
        out_specs=pl.BlockSpec((block_n, f), lambda i: (i, 0)),
        out_shape=jax.ShapeDtypeStruct((n, f), jnp.float32),
    )(z, b)


def _att_mat(att, heads, ch):
    a = att.reshape(heads, ch)
    eye = jnp.eye(heads, dtype=a.dtype)
    return (a[:, :, None] * eye[:, None, :]).reshape(heads * ch, heads)


def _gat_layer(h, asrc, adst, src, dst, heads, block_e_small, block_e_big):
    e = _edge_logits(asrc[src], adst[dst], block_e_small)
    m = jax.ops.segment_max(e, dst, num_segments=_N)
    m = jnp.where(jnp.isfinite(m), m, 0.0)
    ex = _edge_exp(e, m[dst], block_e_small)
    den = jax.ops.segment_sum(ex, dst, num_segments=_N)
    msg = _edge_msg(h[src], ex, den[dst], block_e_big)
    return jax.ops.segment_sum(msg, dst, num_segments=_N)


def kernel(x, edge_index, W1, att_src1, att_dst1, b1, W2, att_src2, att_dst2, b2):
    src = edge_index[0]
    dst = edge_index[1]

    a1s = _att_mat(att_src1, _HEADS, _HID)
    a1d = _att_mat(att_dst1, _HEADS, _HID)
    a2s = _att_mat(att_src2, 1, _OUT)
    a2d = _att_mat(att_dst2, 1, _OUT)

    h1, asrc1, adst1 = _proj1(x, W1.T, a1s, a1d, 1000)
    agg1 = _gat_layer(h1, asrc1, adst1, src, dst, _HEADS, 8000, 2000)

    g, asrc2, adst2 = _proj2(agg1, b1.reshape(1, -1), W2.T, a2s, a2d, 1000)
    agg2 = _gat_layer(g, asrc2, adst2, src, dst, 1, 8000, 2000)

    return _log_softmax(agg2, b2.reshape(1, -1), 1000)
